# baseline (speedup 1.0000x reference)
import functools

import jax
import jax.numpy as jnp
from jax import lax
from jax.experimental import pallas as pl
from jax.experimental.pallas import tpu as pltpu
from jax.experimental.pallas import tpu_sc as plsc

NC, NS = 2, 16
NW = NC * NS
RB = 40
L = 16
DP = 384


def _repack(table, vocab, dim):
    n_blocks = vocab // RB
    orows = RB * DP // 128
    nfull = dim // L
    tail = dim - L

    @functools.partial(
        pl.kernel,
        out_type=jax.ShapeDtypeStruct((vocab * DP // 128, 128), jnp.float32),
        mesh=plsc.VectorSubcoreMesh(core_axis_name="c", subcore_axis_name="s"),
        compiler_params=pltpu.CompilerParams(use_tc_tiling_on_sc=True),
        scratch_types=[
            pltpu.VMEM((RB, dim), jnp.float32),
            pltpu.VMEM((RB, dim), jnp.float32),
            pltpu.VMEM((orows, 128), jnp.float32),
            pltpu.SemaphoreType.DMA,
            pltpu.SemaphoreType.DMA,
        ],
    )
    def k(tbl_hbm, out_hbm, in0, in1, lin, s0, s1):
        wid = lax.axis_index("s") * NC + lax.axis_index("c")

        def read(b, buf, sem):
            return pltpu.make_async_copy(tbl_hbm.at[pl.ds(RB * b, RB)], buf, sem)

        def proc(b, buf):
            for l in range(RB):
                for t in range(nfull):
                    f = DP * l + t * L
                    lin[f // 128, pl.ds(f % 128, L)] = buf[l, pl.ds(t * L, L)]
                f = DP * l + tail
                lin[f // 128, pl.ds(f % 128, L)] = buf[l, pl.ds(tail, L)]
            pltpu.sync_copy(lin, out_hbm.at[pl.ds(orows * b, orows)])

        read(wid, in0, s0).start()

        def body(t, carry):
            b0 = wid + NW * 2 * t
            b1, b2 = b0 + NW, b0 + 2 * NW

            @pl.when(b1 < n_blocks)
            def _():
                read(b1, in1, s1).start()

            read(b0, in0, s0).wait()
            proc(b0, in0)

            @pl.when(b2 < n_blocks)
            def _():
                read(b2, in0, s0).start()

            @pl.when(b1 < n_blocks)
            def _():
                read(b1, in1, s1).wait()
                proc(b1, in1)

            return carry

        n_pairs = (n_blocks - wid + 2 * NW - 1) // (2 * NW)
        lax.fori_loop(0, n_pairs, body, 0)

    return k(table)


def _gather_emit(idxp, tblr, batch, seq, dim):
    n_chunks = batch // NW

    @functools.partial(
        pl.kernel,
        out_type=jax.ShapeDtypeStruct((batch, seq, dim), jnp.float32),
        mesh=plsc.VectorSubcoreMesh(core_axis_name="c", subcore_axis_name="s"),
        compiler_params=pltpu.CompilerParams(use_tc_tiling_on_sc=True),
        scratch_types=[
            pltpu.VMEM((n_chunks, seq), jnp.int32),
            pltpu.VMEM((3, seq), jnp.int32),
            pltpu.VMEM((3, seq), jnp.int32),
            pltpu.VMEM((seq, 128), jnp.float32),
            pltpu.VMEM((seq, 128), jnp.float32),
            pltpu.VMEM((seq, dim), jnp.float32),
            pltpu.VMEM((seq, dim), jnp.float32),
            [pltpu.SemaphoreType.DMA] * 3,
            [pltpu.SemaphoreType.DMA] * 3,
            pltpu.SemaphoreType.DMA,
            pltpu.SemaphoreType.DMA,
        ],
    )
    def k(idx_hbm, tbl_hbm, out_hbm, idx_v, i3a, i3b, s2a, s2b, im0, im1,
          ga, gb_, o0, o1):
        wid = lax.axis_index("s") * NC + lax.axis_index("c")
        pltpu.sync_copy(idx_hbm.at[pl.ds(wid * n_chunks, n_chunks)], idx_v)

        def expand(j, i3):
            for lo in (0, seq - L):
                v = idx_v.at[j][pl.ds(lo, L)]
                b3 = 3 * v
                for ct in range(3):
                    i3[ct, pl.ds(lo, L)] = b3 + ct

        def tiles(img, s2):
            return [
                img.at[:, pl.ds(0, 128)],
                img.at[:, pl.ds(128, 128)],
                s2,
            ]

        def gathers(i3, img, s2, sem):
            for ct, dst in enumerate(tiles(img, s2)):
                pltpu.make_async_copy(
                    tbl_hbm.at[i3.at[ct]], dst, sem[ct]
                ).start()

        def gwait(i3, img, s2, sem):
            for ct, dst in enumerate(tiles(img, s2)):
                pltpu.make_async_copy(
                    tbl_hbm.at[i3.at[ct]], dst, sem[ct]
                ).wait()

        def assemble(j, s2, img, osem):
            for l in range(seq):
                img[l, pl.ds(256, L)] = s2[l, pl.ds(0, L)]
                img[l, pl.ds(dim - L, L)] = s2[l, pl.ds(dim - L - 256, L)]
                img[l, pl.ds(272, L)] = s2[l, pl.ds(16, L)]
            pltpu.make_async_copy(
                img, out_hbm.at[wid * n_chunks + j], osem
            ).start()

        def owait(img, osem):
            pltpu.make_async_copy(img, out_hbm.at[0], osem).wait()

        expand(0, i3a)
        gathers(i3a, im0, s2a, ga)

        def body(t, carry):
            j = 2 * t

            @pl.when(t > 0)
            def _():
                owait(im1, o1)

            expand(j + 1, i3b)
            gathers(i3b, im1, s2b, gb_)

            gwait(i3a, im0, s2a, ga)
            assemble(j, s2a, im0, o0)

            @pl.when(t < n_chunks // 2 - 1)
            def _():
                owait(im0, o0)
                expand(j + 2, i3a)
                gathers(i3a, im0, s2a, ga)

            gwait(i3b, im1, s2b, gb_)
            assemble(j + 1, s2b, im1, o1)
            return carry

        lax.fori_loop(0, n_chunks // 2, body, 0)
        owait(im0, o0)
        owait(im1, o1)

    return k(idxp, tblr)


def kernel(indices, table):
    batch, seq = indices.shape
    vocab, dim = table.shape
    assert batch % NW == 0 and vocab % RB == 0 and 256 < dim <= 300
    tblr = jnp.pad(table, ((0, 0), (0, DP - dim))).reshape(3 * vocab, 128)
    return _gather_emit(indices, tblr, batch, seq, dim)


# speedup vs baseline: 1.8202x; 1.8202x over previous
import functools

import jax
import jax.numpy as jnp
from jax import lax
from jax.experimental import pallas as pl
from jax.experimental.pallas import tpu as pltpu
from jax.experimental.pallas import tpu_sc as plsc

NC, NS = 2, 16
NW = NC * NS
RB = 40
L = 16
DP = 384


def _repack(table, vocab, dim):
    n_blocks = vocab // RB
    orows = RB * DP // 128
    nfull = dim // L
    tail = dim - L

    @functools.partial(
        pl.kernel,
        out_type=jax.ShapeDtypeStruct((vocab * DP // 128, 128), jnp.float32),
        mesh=plsc.VectorSubcoreMesh(core_axis_name="c", subcore_axis_name="s"),
        compiler_params=pltpu.CompilerParams(use_tc_tiling_on_sc=True),
        scratch_types=[
            pltpu.VMEM((RB, dim), jnp.float32),
            pltpu.VMEM((RB, dim), jnp.float32),
            pltpu.VMEM((orows, 128), jnp.float32),
            pltpu.VMEM((orows, 128), jnp.float32),
            pltpu.SemaphoreType.DMA,
            pltpu.SemaphoreType.DMA,
            pltpu.SemaphoreType.DMA,
            pltpu.SemaphoreType.DMA,
        ],
    )
    def k(tbl_hbm, out_hbm, in0, in1, lin0, lin1, s0, s1, w0, w1):
        wid = lax.axis_index("s") * NC + lax.axis_index("c")

        def read(b, buf, sem):
            return pltpu.make_async_copy(tbl_hbm.at[pl.ds(RB * b, RB)], buf, sem)

        def wwait(lin, wsem):
            pltpu.make_async_copy(lin, out_hbm.at[pl.ds(0, orows)], wsem).wait()

        def proc(b, buf, lin, wsem):
            for l in range(RB):
                for t in range(nfull):
                    f = DP * l + t * L
                    lin[f // 128, pl.ds(f % 128, L)] = buf[l, pl.ds(t * L, L)]
                f = DP * l + tail
                lin[f // 128, pl.ds(f % 128, L)] = buf[l, pl.ds(tail, L)]
            pltpu.make_async_copy(
                lin, out_hbm.at[pl.ds(orows * b, orows)], wsem
            ).start()

        read(wid, in0, s0).start()

        def body(t, carry):
            b0 = wid + NW * 2 * t
            b1, b2 = b0 + NW, b0 + 2 * NW

            @pl.when(b1 < n_blocks)
            def _():
                read(b1, in1, s1).start()

            read(b0, in0, s0).wait()

            @pl.when(t > 0)
            def _():
                wwait(lin0, w0)

            proc(b0, in0, lin0, w0)

            @pl.when(b2 < n_blocks)
            def _():
                read(b2, in0, s0).start()

            @pl.when(b1 < n_blocks)
            def _():
                read(b1, in1, s1).wait()

                @pl.when(t > 0)
                def _():
                    wwait(lin1, w1)

                proc(b1, in1, lin1, w1)

            return carry

        n_pairs = (n_blocks - wid + 2 * NW - 1) // (2 * NW)
        lax.fori_loop(0, n_pairs, body, 0)
        wwait(lin0, w0)
        wwait(lin1, w1)

    return k(table)


def _gather_emit(idxp, tblr, batch, seq, dim):
    n_chunks = batch // NW

    @functools.partial(
        pl.kernel,
        out_type=jax.ShapeDtypeStruct((batch, seq, dim), jnp.float32),
        mesh=plsc.VectorSubcoreMesh(core_axis_name="c", subcore_axis_name="s"),
        compiler_params=pltpu.CompilerParams(use_tc_tiling_on_sc=True),
        scratch_types=[
            pltpu.VMEM((n_chunks, seq), jnp.int32),
            pltpu.VMEM((3, seq), jnp.int32),
            pltpu.VMEM((3, seq), jnp.int32),
            pltpu.VMEM((seq, 128), jnp.float32),
            pltpu.VMEM((seq, 128), jnp.float32),
            pltpu.VMEM((seq, dim), jnp.float32),
            pltpu.VMEM((seq, dim), jnp.float32),
            [pltpu.SemaphoreType.DMA] * 3,
            [pltpu.SemaphoreType.DMA] * 3,
            pltpu.SemaphoreType.DMA,
            pltpu.SemaphoreType.DMA,
        ],
    )
    def k(idx_hbm, tbl_hbm, out_hbm, idx_v, i3a, i3b, s2a, s2b, im0, im1,
          ga, gb_, o0, o1):
        wid = lax.axis_index("s") * NC + lax.axis_index("c")
        pltpu.sync_copy(idx_hbm.at[pl.ds(wid * n_chunks, n_chunks)], idx_v)

        def expand(j, i3):
            for lo in (0, seq - L):
                v = idx_v.at[j][pl.ds(lo, L)]
                b3 = 3 * v
                for ct in range(3):
                    i3[ct, pl.ds(lo, L)] = b3 + ct

        def tiles(img, s2):
            return [
                img.at[:, pl.ds(0, 128)],
                img.at[:, pl.ds(128, 128)],
                s2,
            ]

        def gathers(i3, img, s2, sem):
            for ct, dst in enumerate(tiles(img, s2)):
                pltpu.make_async_copy(
                    tbl_hbm.at[i3.at[ct]], dst, sem[ct]
                ).start()

        def gwait(i3, img, s2, sem):
            for ct, dst in enumerate(tiles(img, s2)):
                pltpu.make_async_copy(
                    tbl_hbm.at[i3.at[ct]], dst, sem[ct]
                ).wait()

        def assemble(j, s2, img, osem):
            for l in range(seq):
                img[l, pl.ds(256, L)] = s2[l, pl.ds(0, L)]
                img[l, pl.ds(dim - L, L)] = s2[l, pl.ds(dim - L - 256, L)]
                img[l, pl.ds(272, L)] = s2[l, pl.ds(16, L)]
            pltpu.make_async_copy(
                img, out_hbm.at[wid * n_chunks + j], osem
            ).start()

        def owait(img, osem):
            pltpu.make_async_copy(img, out_hbm.at[0], osem).wait()

        expand(0, i3a)
        gathers(i3a, im0, s2a, ga)

        def body(t, carry):
            j = 2 * t

            @pl.when(t > 0)
            def _():
                owait(im1, o1)

            expand(j + 1, i3b)
            gathers(i3b, im1, s2b, gb_)

            gwait(i3a, im0, s2a, ga)
            assemble(j, s2a, im0, o0)

            @pl.when(t < n_chunks // 2 - 1)
            def _():
                owait(im0, o0)
                expand(j + 2, i3a)
                gathers(i3a, im0, s2a, ga)

            gwait(i3b, im1, s2b, gb_)
            assemble(j + 1, s2b, im1, o1)
            return carry

        lax.fori_loop(0, n_chunks // 2, body, 0)
        owait(im0, o0)
        owait(im1, o1)

    return k(idxp, tblr)


def kernel(indices, table):
    batch, seq = indices.shape
    vocab, dim = table.shape
    assert batch % NW == 0 and vocab % RB == 0 and 256 < dim <= 300
    tblr = _repack(table, vocab, dim)
    return _gather_emit(indices, tblr, batch, seq, dim)


# speedup vs baseline: 1.8306x; 1.0057x over previous
import functools

import jax
import jax.numpy as jnp
from jax import lax
from jax.experimental import pallas as pl
from jax.experimental.pallas import tpu as pltpu
from jax.experimental.pallas import tpu_sc as plsc

NC, NS = 2, 16
NW = NC * NS
RB = 80
L = 16
DP = 384


def _repack(table, vocab, dim):
    n_blocks = vocab // RB
    orows = RB * DP // 128
    nfull = dim // L
    tail = dim - L

    @functools.partial(
        pl.kernel,
        out_type=jax.ShapeDtypeStruct((vocab * DP // 128, 128), jnp.float32),
        mesh=plsc.VectorSubcoreMesh(core_axis_name="c", subcore_axis_name="s"),
        compiler_params=pltpu.CompilerParams(use_tc_tiling_on_sc=True),
        scratch_types=[
            pltpu.VMEM((RB, dim), jnp.float32),
            pltpu.VMEM((RB, dim), jnp.float32),
            pltpu.VMEM((orows, 128), jnp.float32),
            pltpu.VMEM((orows, 128), jnp.float32),
            pltpu.SemaphoreType.DMA,
            pltpu.SemaphoreType.DMA,
            pltpu.SemaphoreType.DMA,
            pltpu.SemaphoreType.DMA,
        ],
    )
    def k(tbl_hbm, out_hbm, in0, in1, lin0, lin1, s0, s1, w0, w1):
        wid = lax.axis_index("s") * NC + lax.axis_index("c")

        def read(b, buf, sem):
            return pltpu.make_async_copy(tbl_hbm.at[pl.ds(RB * b, RB)], buf, sem)

        def wwait(lin, wsem):
            pltpu.make_async_copy(lin, out_hbm.at[pl.ds(0, orows)], wsem).wait()

        def proc(b, buf, lin, wsem):
            for l in range(RB):
                for t in range(nfull):
                    f = DP * l + t * L
                    lin[f // 128, pl.ds(f % 128, L)] = buf[l, pl.ds(t * L, L)]
                f = DP * l + tail
                lin[f // 128, pl.ds(f % 128, L)] = buf[l, pl.ds(tail, L)]
            pltpu.make_async_copy(
                lin, out_hbm.at[pl.ds(orows * b, orows)], wsem
            ).start()

        read(wid, in0, s0).start()

        def body(t, carry):
            b0 = wid + NW * 2 * t
            b1, b2 = b0 + NW, b0 + 2 * NW

            @pl.when(b1 < n_blocks)
            def _():
                read(b1, in1, s1).start()

            read(b0, in0, s0).wait()

            @pl.when(t > 0)
            def _():
                wwait(lin0, w0)

            proc(b0, in0, lin0, w0)

            @pl.when(b2 < n_blocks)
            def _():
                read(b2, in0, s0).start()

            @pl.when(b1 < n_blocks)
            def _():
                read(b1, in1, s1).wait()

                @pl.when(t > 0)
                def _():
                    wwait(lin1, w1)

                proc(b1, in1, lin1, w1)

            return carry

        n_pairs = (n_blocks - wid + 2 * NW - 1) // (2 * NW)
        lax.fori_loop(0, n_pairs, body, 0)
        wwait(lin0, w0)
        wwait(lin1, w1)

    return k(table)


def _gather_emit(idxp, tblr, batch, seq, dim):
    n_chunks = batch // NW

    @functools.partial(
        pl.kernel,
        out_type=jax.ShapeDtypeStruct((batch, seq, dim), jnp.float32),
        mesh=plsc.VectorSubcoreMesh(core_axis_name="c", subcore_axis_name="s"),
        compiler_params=pltpu.CompilerParams(use_tc_tiling_on_sc=True),
        scratch_types=[
            pltpu.VMEM((n_chunks, seq), jnp.int32),
            pltpu.VMEM((3, seq), jnp.int32),
            pltpu.VMEM((3, seq), jnp.int32),
            pltpu.VMEM((seq, 128), jnp.float32),
            pltpu.VMEM((seq, 128), jnp.float32),
            pltpu.VMEM((seq, dim), jnp.float32),
            pltpu.VMEM((seq, dim), jnp.float32),
            [pltpu.SemaphoreType.DMA] * 3,
            [pltpu.SemaphoreType.DMA] * 3,
            pltpu.SemaphoreType.DMA,
            pltpu.SemaphoreType.DMA,
        ],
    )
    def k(idx_hbm, tbl_hbm, out_hbm, idx_v, i3a, i3b, s2a, s2b, im0, im1,
          ga, gb_, o0, o1):
        wid = lax.axis_index("s") * NC + lax.axis_index("c")
        pltpu.sync_copy(idx_hbm.at[pl.ds(wid * n_chunks, n_chunks)], idx_v)

        def expand(j, i3):
            for lo in (0, seq - L):
                v = idx_v.at[j][pl.ds(lo, L)]
                b3 = 3 * v
                for ct in range(3):
                    i3[ct, pl.ds(lo, L)] = b3 + ct

        def tiles(img, s2):
            return [
                img.at[:, pl.ds(0, 128)],
                img.at[:, pl.ds(128, 128)],
                s2,
            ]

        def gathers(i3, img, s2, sem):
            for ct, dst in enumerate(tiles(img, s2)):
                pltpu.make_async_copy(
                    tbl_hbm.at[i3.at[ct]], dst, sem[ct]
                ).start()

        def gwait(i3, img, s2, sem):
            for ct, dst in enumerate(tiles(img, s2)):
                pltpu.make_async_copy(
                    tbl_hbm.at[i3.at[ct]], dst, sem[ct]
                ).wait()

        def assemble(j, s2, img, osem):
            for l in range(seq):
                img[l, pl.ds(256, L)] = s2[l, pl.ds(0, L)]
                img[l, pl.ds(dim - L, L)] = s2[l, pl.ds(dim - L - 256, L)]
                img[l, pl.ds(272, L)] = s2[l, pl.ds(16, L)]
            pltpu.make_async_copy(
                img, out_hbm.at[wid * n_chunks + j], osem
            ).start()

        def owait(img, osem):
            pltpu.make_async_copy(img, out_hbm.at[0], osem).wait()

        expand(0, i3a)
        gathers(i3a, im0, s2a, ga)

        def body(t, carry):
            j = 2 * t

            @pl.when(t > 0)
            def _():
                owait(im1, o1)

            expand(j + 1, i3b)
            gathers(i3b, im1, s2b, gb_)

            gwait(i3a, im0, s2a, ga)
            assemble(j, s2a, im0, o0)

            @pl.when(t < n_chunks // 2 - 1)
            def _():
                owait(im0, o0)
                expand(j + 2, i3a)
                gathers(i3a, im0, s2a, ga)

            gwait(i3b, im1, s2b, gb_)
            assemble(j + 1, s2b, im1, o1)
            return carry

        lax.fori_loop(0, n_chunks // 2, body, 0)
        owait(im0, o0)
        owait(im1, o1)

    return k(idxp, tblr)


def kernel(indices, table):
    batch, seq = indices.shape
    vocab, dim = table.shape
    assert batch % NW == 0 and vocab % RB == 0 and 256 < dim <= 300
    tblr = _repack(table, vocab, dim)
    return _gather_emit(indices, tblr, batch, seq, dim)
